# Initial kernel scaffold; baseline (speedup 1.0000x reference)
import jax
import jax.numpy as jnp
from jax.experimental import pallas as pl


def kernel(x, edge_index, W, b):
    raise NotImplementedError("write your pallas kernel here")


# speedup vs baseline: 29.2056x; 29.2056x over previous
import functools

import jax
import jax.numpy as jnp
from jax import lax
from jax.experimental import pallas as pl
from jax.experimental.pallas import tpu as pltpu
from jax.experimental.pallas import tpu_sc as plsc

N = 10000
E = 320000
F = 128
NPAD = 10240
NC, NS, L = 2, 16, 16
NW = NC * NS
EPW = E // NW
K = 100
CHN = EPW // K
RPT = NPAD // NS

_mesh = plsc.VectorSubcoreMesh(core_axis_name="c", subcore_axis_name="s")


@functools.partial(
    pl.kernel,
    out_type=jax.ShapeDtypeStruct((NC, NPAD), jnp.float32),
    mesh=_mesh,
    scratch_types=[
        pltpu.VMEM((CHN, K), jnp.int32),
        pltpu.VMEM((112,), jnp.float32),
        pltpu.VMEM((RPT,), jnp.float32),
        pltpu.VMEM_SHARED((NPAD,), jnp.float32),
    ],
)
def _sc_deg(col_hbm, deg_out, cidx, ones, stage, acc):
    c = lax.axis_index("c")
    s = lax.axis_index("s")
    wid = s * NC + c
    for i in range(RPT // L):
        stage[pl.ds(i * L, L)] = jnp.zeros((L,), jnp.float32)
    for i in range(112 // L):
        ones[pl.ds(i * L, L)] = jnp.ones((L,), jnp.float32)
    pltpu.sync_copy(stage, acc.at[pl.ds(s * RPT, RPT)])
    plsc.subcore_barrier()
    pltpu.sync_copy(col_hbm.at[wid], cidx)

    def body(j, carry):
        pltpu.sync_copy(ones.at[pl.ds(0, K)], acc.at[cidx.at[j]], add=True)
        return carry

    lax.fori_loop(0, CHN, body, 0)
    plsc.subcore_barrier()
    pltpu.sync_copy(acc.at[pl.ds(s * RPT, RPT)], stage)
    pltpu.sync_copy(stage, deg_out.at[c, pl.ds(s * RPT, RPT)])


@functools.partial(
    pl.kernel,
    out_type=jax.ShapeDtypeStruct((NC, NPAD, F), jnp.float32),
    mesh=_mesh,
    scratch_types=[
        pltpu.VMEM((CHN, K), jnp.int32),
        pltpu.VMEM((CHN, K), jnp.int32),
        pltpu.VMEM((K, F), jnp.float32),
        pltpu.VMEM((64, F), jnp.float32),
        pltpu.VMEM_SHARED((NPAD, F), jnp.float32),
        pltpu.SemaphoreType.DMA,
    ],
)
def _sc_edge(row_hbm, col_hbm, y_hbm, out_hbm, ridx, cidx, rows, stage, acc,
             sem):
    c = lax.axis_index("c")
    s = lax.axis_index("s")
    wid = s * NC + c
    for i in range(64):
        for jj in range(F // L):
            stage[i, pl.ds(jj * L, L)] = jnp.zeros((L,), jnp.float32)
    for t in range(RPT // 64):
        pltpu.sync_copy(stage, acc.at[pl.ds(s * RPT + t * 64, 64)])
    plsc.subcore_barrier()
    pltpu.sync_copy(row_hbm.at[wid], ridx)
    pltpu.sync_copy(col_hbm.at[wid], cidx)

    def body(j, carry):
        pltpu.async_copy(y_hbm.at[ridx.at[j]], rows, sem).wait()
        pltpu.sync_copy(rows, acc.at[cidx.at[j]], add=True)
        return carry

    lax.fori_loop(0, CHN, body, 0)
    plsc.subcore_barrier()
    for t in range(RPT // 64):
        pltpu.sync_copy(acc.at[pl.ds(s * RPT + t * 64, 64)], stage)
        pltpu.sync_copy(stage, out_hbm.at[c, pl.ds(s * RPT + t * 64, 64)])


def _lin_body(x_ref, w_ref, d_ref, y_ref):
    deg = d_ref[0] + d_ref[1] + 1.0
    dis = lax.rsqrt(deg)
    y_ref[...] = jnp.dot(x_ref[...], w_ref[...],
                         preferred_element_type=jnp.float32) * dis


def _epi_body(p_ref, y_ref, d_ref, b_ref, o_ref):
    deg = d_ref[0] + d_ref[1] + 1.0
    dis = lax.rsqrt(deg)
    o_ref[...] = dis * (p_ref[0] + p_ref[1] + y_ref[...]) + b_ref[...]


_RB = 1280


def _tc_linear(xp, W, degp):
    return pl.pallas_call(
        _lin_body,
        grid=(NPAD // _RB,),
        in_specs=[
            pl.BlockSpec((_RB, F), lambda i: (i, 0)),
            pl.BlockSpec((F, F), lambda i: (0, 0)),
            pl.BlockSpec((NC, _RB, 1), lambda i: (0, i, 0)),
        ],
        out_specs=pl.BlockSpec((_RB, F), lambda i: (i, 0)),
        out_shape=jax.ShapeDtypeStruct((NPAD, F), jnp.float32),
    )(xp, W, degp)


def _tc_epilogue(parts, y, degp, b2):
    return pl.pallas_call(
        _epi_body,
        grid=(NPAD // _RB,),
        in_specs=[
            pl.BlockSpec((NC, _RB, F), lambda i: (0, i, 0)),
            pl.BlockSpec((_RB, F), lambda i: (i, 0)),
            pl.BlockSpec((NC, _RB, 1), lambda i: (0, i, 0)),
            pl.BlockSpec((1, F), lambda i: (0, 0)),
        ],
        out_specs=pl.BlockSpec((_RB, F), lambda i: (i, 0)),
        out_shape=jax.ShapeDtypeStruct((NPAD, F), jnp.float32),
    )(parts, y, degp, b2)


def kernel(x, edge_index, W, b):
    row3 = edge_index[0].reshape(NW, CHN, K)
    col3 = edge_index[1].reshape(NW, CHN, K)
    deg_parts = _sc_deg(col3)
    degp = deg_parts.reshape(NC, NPAD, 1)
    xp = jnp.pad(x, ((0, NPAD - N), (0, 0)))
    y = _tc_linear(xp, W, degp)
    parts = _sc_edge(row3, col3, y)
    out = _tc_epilogue(parts, y, degp, b.reshape(1, F))
    return out[:N]
